# baseline (speedup 1.0000x reference)
import functools

import jax
import jax.numpy as jnp
from jax import lax
from jax.experimental import pallas as pl
from jax.experimental.pallas import tpu as pltpu
from jax.experimental.pallas import tpu_sc as plsc


def kernel(x, embedding_table):
    seq_len = x.shape[1]
    table_rows, d_model = embedding_table.shape

    info = plsc.get_sparse_core_info()
    nc, ns = info.num_cores, info.num_subcores
    nw = nc * ns
    rows_per_w = seq_len // nw
    row_elems = seq_len * d_model

    mesh = plsc.VectorSubcoreMesh(core_axis_name="c", subcore_axis_name="s")

    nbuf = 4
    pieces = 16
    chunk = row_elems // pieces
    nchunks = rows_per_w * pieces

    @functools.partial(
        pl.kernel,
        mesh=mesh,
        out_type=jax.ShapeDtypeStruct((seq_len * seq_len * d_model,), jnp.float32),
        scratch_types=[
            pltpu.VMEM((nbuf, chunk), jnp.float32),
            pltpu.SemaphoreType.DMA((nbuf,)),
            pltpu.SemaphoreType.DMA((nbuf,)),
        ],
    )
    def copy_kernel(table_hbm, out_hbm, buf, sem_in, sem_out):
        cid = lax.axis_index("c")
        sid = lax.axis_index("s")
        wid = sid * nc + cid
        row0 = wid * rows_per_w

        def src_of(k):
            r = lax.div(k, pieces)
            p = lax.rem(k, pieces)
            return pl.multiple_of(
                (seq_len - row0 - r) * d_model + p * chunk, 128
            )

        def dst_of(k):
            r = lax.div(k, pieces)
            p = lax.rem(k, pieces)
            return pl.multiple_of((row0 + r) * row_elems + p * chunk, 128)

        def start_in(k, b):
            pltpu.make_async_copy(
                table_hbm.at[pl.ds(src_of(k), chunk)], buf.at[b], sem_in.at[b]
            ).start()

        def wait_in(k, b):
            pltpu.make_async_copy(
                table_hbm.at[pl.ds(src_of(k), chunk)], buf.at[b], sem_in.at[b]
            ).wait()

        def start_out(k, b):
            pltpu.make_async_copy(
                buf.at[b], out_hbm.at[pl.ds(dst_of(k), chunk)], sem_out.at[b]
            ).start()

        def wait_out(k, b):
            pltpu.make_async_copy(
                buf.at[b], out_hbm.at[pl.ds(dst_of(k), chunk)], sem_out.at[b]
            ).wait()

        for b in range(nbuf):
            start_in(b, b)

        def step(k4, carry):
            for b in range(nbuf):
                k = k4 * nbuf + b
                wait_in(k, b)
                start_out(k, b)
                nxt = k + nbuf

                @pl.when(nxt < nchunks)
                def _refill():
                    wait_out(k, b)
                    start_in(nxt, b)

                @pl.when(nxt >= nchunks)
                def _final_drain():
                    wait_out(k, b)

            return carry

        lax.fori_loop(0, nchunks // nbuf, step, 0, unroll=False)

    flat = copy_kernel(embedding_table.reshape(-1))
    return flat.reshape(seq_len, seq_len, d_model)
